# baseline (speedup 1.0000x reference)
import functools

import jax
import jax.numpy as jnp
from jax import lax
from jax.experimental import pallas as pl
from jax.experimental.pallas import tpu as pltpu
from jax.experimental.pallas import tpu_sc as plsc

B = 16384
F = 32
V = 1_000_000
NC = 2
NS = 16
NW = NC * NS
BPW = B // NW
CHUNK = 128
NCHUNK = BPW // CHUNK
GROUPS = BPW // 16
VC = 65536
NVC = -(-V // VC)
FR = 8
NFR = F // FR
FLAT = NVC * VC * F


def _bias_relayout_body(in_ref, out_ref):
    out_ref[...] = in_ref[0, :]


def _relayout_body(in_ref, out_ref):
    for p in range(FR // 2):
        lo = jax.lax.bitcast_convert_type(
            in_ref[2 * p, :].astype(jnp.bfloat16), jnp.uint16).astype(jnp.uint32)
        hi = jax.lax.bitcast_convert_type(
            in_ref[2 * p + 1, :].astype(jnp.bfloat16), jnp.uint16).astype(jnp.uint32)
        out_ref[pl.ds(p * VC, VC)] = lo | (hi << 16)


def _relayout(tableT):
    return pl.pallas_call(
        _relayout_body,
        grid=(NVC, NFR),
        in_specs=[pl.BlockSpec((FR, VC), lambda i, t: (t, i))],
        out_specs=pl.BlockSpec((FR * VC // 2,), lambda i, t: (i * NFR + t,)),
        out_shape=jax.ShapeDtypeStruct((FLAT // 2,), jnp.uint32),
    )(tableT)


def _bias_relayout(biasT):
    return pl.pallas_call(
        _bias_relayout_body,
        grid=(NVC,),
        in_specs=[pl.BlockSpec((1, VC), lambda i: (0, i))],
        out_specs=pl.BlockSpec((VC,), lambda i: (i,)),
        out_shape=jax.ShapeDtypeStruct((NVC * VC,), jnp.float32),
    )(biasT)


def _mf_body(uid_hbm, iid_hbm, uflat_hbm, ubias_hbm, iflat_hbm, ibias_hbm,
             out_hbm, uid_v, iid_v, uidx_v, iidx_v, ucols_v, icols_v,
             ub_v, ib_v, out_v, sem):
    wid = lax.axis_index("s") * NC + lax.axis_index("c")
    base = wid * BPW

    pltpu.sync_copy(uid_hbm.at[pl.ds(base, BPW)], uid_v)
    pltpu.sync_copy(iid_hbm.at[pl.ds(base, BPW)], iid_v)

    for c in range(NCHUNK):
        sl = pl.ds(c * CHUNK, CHUNK)
        for j in range(CHUNK // 16):
            jsl = pl.ds(c * CHUNK + j * 16, 16)
            osl = pl.ds(j * 16, 16)
            uv = uid_v[jsl]
            iv = iid_v[jsl]
            ub_base = ((uv >> 16) << 20) + (uv & 0xFFFF)
            ib_base = ((iv >> 16) << 20) + (iv & 0xFFFF)
            for p in range(F // 2):
                uidx_v[p, osl] = ub_base + (p << 16)
                iidx_v[p, osl] = ib_base + (p << 16)
        copies = [
            pltpu.async_copy(ubias_hbm.at[uid_v.at[sl]], ub_v.at[sl], sem),
            pltpu.async_copy(ibias_hbm.at[iid_v.at[sl]], ib_v.at[sl], sem),
        ]
        for p in range(F // 2):
            copies.append(pltpu.async_copy(
                uflat_hbm.at[uidx_v.at[p]], ucols_v.at[p].at[sl], sem))
            copies.append(pltpu.async_copy(
                iflat_hbm.at[iidx_v.at[p]], icols_v.at[p].at[sl], sem))
        for cp in copies:
            cp.wait()

    def group(g, carry):
        i0 = g * 16
        gsl = pl.ds(i0, 16)
        acc = ub_v[gsl] + ib_v[gsl]
        for p in range(F // 2):
            ue, uo = plsc.unpack(plsc.bitcast(ucols_v[p, gsl], jnp.bfloat16),
                                 format=plsc.PackFormat.INTERLEAVED)
            ie, io = plsc.unpack(plsc.bitcast(icols_v[p, gsl], jnp.bfloat16),
                                 format=plsc.PackFormat.INTERLEAVED)
            acc = acc + ue * ie + uo * io
        out_v[gsl] = acc
        return carry

    lax.fori_loop(0, GROUPS, group, 0)

    pltpu.sync_copy(out_v, out_hbm.at[pl.ds(base, BPW)])


@jax.jit
def kernel(user_id, item_id, user_embedding, user_bias, item_embedding, item_bias):
    run = pl.kernel(
        _mf_body,
        out_type=jax.ShapeDtypeStruct((B,), jnp.float32),
        mesh=plsc.VectorSubcoreMesh(core_axis_name="c", subcore_axis_name="s"),
        compiler_params=pltpu.CompilerParams(
            needs_layout_passes=False, use_tc_tiling_on_sc=False),
        scratch_types=[
            pltpu.VMEM((BPW,), jnp.int32),
            pltpu.VMEM((BPW,), jnp.int32),
            pltpu.VMEM((F // 2, CHUNK), jnp.int32),
            pltpu.VMEM((F // 2, CHUNK), jnp.int32),
            pltpu.VMEM((F // 2, BPW), jnp.uint32),
            pltpu.VMEM((F // 2, BPW), jnp.uint32),
            pltpu.VMEM((BPW,), jnp.float32),
            pltpu.VMEM((BPW,), jnp.float32),
            pltpu.VMEM((BPW,), jnp.float32),
            pltpu.SemaphoreType.DMA,
        ],
    )
    uflat = _relayout(user_embedding.T)
    iflat = _relayout(item_embedding.T)
    return run(user_id, item_id, uflat, _bias_relayout(user_bias.T),
               iflat, _bias_relayout(item_bias.T))


# speedup vs baseline: 1.7312x; 1.7312x over previous
import functools

import jax
import jax.numpy as jnp
from jax import lax
from jax.experimental import pallas as pl
from jax.experimental.pallas import tpu as pltpu
from jax.experimental.pallas import tpu_sc as plsc

B = 16384
F = 32
V = 1_000_000
NC = 2
NS = 16
NW = NC * NS
BPW = B // NW
CHUNK = 128
NCHUNK = BPW // CHUNK
GROUPS = BPW // 16
VC = 65536
NVC = -(-V // VC)
FR = 8
NFR = F // FR
FLAT = NVC * VC * F


def _bias_relayout_body(in_ref, out_ref):
    out_ref[...] = in_ref[0, :]


def _relayout_body(in_ref, out_ref):
    for f in range(FR):
        out_ref[pl.ds(f * VC, VC)] = in_ref[f, :]


def _relayout(tableT):
    return pl.pallas_call(
        _relayout_body,
        grid=(NVC, NFR),
        in_specs=[pl.BlockSpec((FR, VC), lambda i, t: (t, i))],
        out_specs=pl.BlockSpec((FR * VC,), lambda i, t: (i * NFR + t,)),
        out_shape=jax.ShapeDtypeStruct((FLAT,), jnp.float32),
    )(tableT)


def _bias_relayout(biasT):
    return pl.pallas_call(
        _bias_relayout_body,
        grid=(NVC,),
        in_specs=[pl.BlockSpec((1, VC), lambda i: (0, i))],
        out_specs=pl.BlockSpec((VC,), lambda i: (i,)),
        out_shape=jax.ShapeDtypeStruct((NVC * VC,), jnp.float32),
    )(biasT)


def _mf_body(uid_hbm, iid_hbm, uflat_hbm, ubias_hbm, iflat_hbm, ibias_hbm,
             out_hbm, uid_v, iid_v, uidx_v, iidx_v, ucols_v, icols_v,
             ub_v, ib_v, out_v, sem):
    wid = lax.axis_index("s") * NC + lax.axis_index("c")
    base = wid * BPW

    pltpu.sync_copy(uid_hbm.at[pl.ds(base, BPW)], uid_v)
    pltpu.sync_copy(iid_hbm.at[pl.ds(base, BPW)], iid_v)

    for c in range(NCHUNK):
        sl = pl.ds(c * CHUNK, CHUNK)
        for j in range(CHUNK // 16):
            jsl = pl.ds(c * CHUNK + j * 16, 16)
            osl = pl.ds(j * 16, 16)
            uv = uid_v[jsl]
            iv = iid_v[jsl]
            ub_base = ((uv >> 16) << 21) + (uv & 0xFFFF)
            ib_base = ((iv >> 16) << 21) + (iv & 0xFFFF)
            for f in range(F):
                uidx_v[f, osl] = ub_base + (f << 16)
                iidx_v[f, osl] = ib_base + (f << 16)
        copies = [
            pltpu.async_copy(ubias_hbm.at[uid_v.at[sl]], ub_v.at[sl], sem),
            pltpu.async_copy(ibias_hbm.at[iid_v.at[sl]], ib_v.at[sl], sem),
        ]
        for f in range(F):
            copies.append(pltpu.async_copy(
                uflat_hbm.at[uidx_v.at[f]], ucols_v.at[f].at[sl], sem))
            copies.append(pltpu.async_copy(
                iflat_hbm.at[iidx_v.at[f]], icols_v.at[f].at[sl], sem))
        for cp in copies:
            cp.wait()

    def group(g, carry):
        i0 = g * 16
        gsl = pl.ds(i0, 16)
        acc = ub_v[gsl] + ib_v[gsl]
        for f in range(F):
            acc = acc + ucols_v[f, gsl] * icols_v[f, gsl]
        out_v[gsl] = acc
        return carry

    lax.fori_loop(0, GROUPS, group, 0)

    pltpu.sync_copy(out_v, out_hbm.at[pl.ds(base, BPW)])


@jax.jit
def kernel(user_id, item_id, user_embedding, user_bias, item_embedding, item_bias):
    run = pl.kernel(
        _mf_body,
        out_type=jax.ShapeDtypeStruct((B,), jnp.float32),
        mesh=plsc.VectorSubcoreMesh(core_axis_name="c", subcore_axis_name="s"),
        compiler_params=pltpu.CompilerParams(
            needs_layout_passes=False, use_tc_tiling_on_sc=False),
        scratch_types=[
            pltpu.VMEM((BPW,), jnp.int32),
            pltpu.VMEM((BPW,), jnp.int32),
            pltpu.VMEM((F, CHUNK), jnp.int32),
            pltpu.VMEM((F, CHUNK), jnp.int32),
            pltpu.VMEM((F, BPW), jnp.float32),
            pltpu.VMEM((F, BPW), jnp.float32),
            pltpu.VMEM((BPW,), jnp.float32),
            pltpu.VMEM((BPW,), jnp.float32),
            pltpu.VMEM((BPW,), jnp.float32),
            pltpu.SemaphoreType.DMA,
        ],
    )
    uflat = _relayout(user_embedding.T)
    iflat = _relayout(item_embedding.T)
    return run(user_id, item_id, uflat, _bias_relayout(user_bias.T),
               iflat, _bias_relayout(item_bias.T))
